# baseline (speedup 1.0000x reference)
import functools

import jax
import jax.numpy as jnp
from jax import lax
from jax.experimental import pallas as pl
from jax.experimental.pallas import tpu as pltpu
from jax.experimental.pallas import tpu_sc as plsc

_NUM_CORES = 2
_NUM_SUBCORES = 16
_NUM_WORKERS = _NUM_CORES * _NUM_SUBCORES
_L = 16

_RADIX_BITS = 11
_NUM_BINS = 1 << _RADIX_BITS
_SHIFTS = (0, _RADIX_BITS, 2 * _RADIX_BITS)
_NSEG = 4


def _vec(val):
  return lax.full((_L,), val, jnp.int32)


def _lsr(x, k):
  if k == 0:
    return x
  return lax.shift_right_logical(x, _vec(k))


def _to_sortable_bits(f):
  b = plsc.bitcast(f, jnp.int32)
  sgn = lax.shift_right_arithmetic(b, _vec(31))
  flip = lax.bitwise_or(sgn, _vec(-(2**31)))
  return lax.bitwise_xor(b, flip)


@functools.partial(jax.jit, static_argnames=())
def _argsort_rows(x):
  rows, n = x.shape
  assert rows % _NUM_WORKERS == 0 and n % (_L * _NSEG) == 0
  rows_per_worker = rows // _NUM_WORKERS
  seg_chunks = n // (_L * _NSEG)
  meta_slots = 512
  hist_chunks = _NUM_BINS // _L

  mesh = plsc.VectorSubcoreMesh(
      core_axis_name="c", subcore_axis_name="s")

  def body(x_hbm, out_hbm, keyf, ping, pong, meta, h0, h1, h2, h3):
    cid = lax.axis_index("c")
    sid = lax.axis_index("s")
    wid = sid * _NUM_CORES + cid
    hsegs = (h0, h1, h2, h3)

    def zero_hists():
      @plsc.parallel_loop(0, hist_chunks, unroll=4)
      def _(j):
        sl = pl.ds(j * _L, _L)
        for h in hsegs:
          h[sl] = _vec(0)

    def prefix_hists():
      def pbody(j, carry):
        sl = pl.ds(j * _L, _L)
        vs = [h[sl] for h in hsegs]
        tot = vs[0]
        for s in range(1, _NSEG):
          tot = tot + vs[s]
        base = plsc.cumsum(tot) - tot + carry
        for s in range(_NSEG):
          hsegs[s][sl] = base
          if s + 1 < _NSEG:
            base = base + vs[s]
        return carry + jnp.sum(tot)
      lax.fori_loop(0, hist_chunks, pbody, jnp.int32(-1))

    def transform_and_count0():
      @plsc.parallel_loop(0, seg_chunks, unroll=2)
      def _(j):
        for s in range(_NSEG):
          sl = pl.ds((s * seg_chunks + j) * _L, _L)
          u = _to_sortable_bits(keyf[sl])
          keyf[sl] = plsc.bitcast(u, jnp.float32)
          d = lax.bitwise_and(u, _vec(_NUM_BINS - 1))
          occ, last = plsc.scan_count(d)
          plsc.addupdate_scatter(hsegs[s], [d], occ, mask=last)

    def count_sweep(src, shift):
      @plsc.parallel_loop(0, seg_chunks, unroll=2)
      def _(j):
        for s in range(_NSEG):
          sl = pl.ds((s * seg_chunks + j) * _L, _L)
          u = plsc.bitcast(plsc.load_gather(keyf, [src[sl]]), jnp.int32)
          d = lax.bitwise_and(_lsr(u, shift), _vec(_NUM_BINS - 1))
          occ, last = plsc.scan_count(d)
          plsc.addupdate_scatter(hsegs[s], [d], occ, mask=last)

    def permute(src, dst, shift):
      iota = lax.iota(jnp.int32, _L)
      win_rounds = meta_slots // _NSEG
      rounds_per_iter = 2

      for w in range(seg_chunks // win_rounds):
        wbase = w * win_rounds

        @plsc.parallel_loop(0, win_rounds, unroll=2)
        def _(j):
          for s in range(_NSEG):
            c = s * seg_chunks + wbase + j
            sl = pl.ds(c * _L, _L)
            if src is None:
              u = plsc.bitcast(keyf[sl], jnp.int32)
            else:
              u = plsc.bitcast(plsc.load_gather(keyf, [src[sl]]), jnp.int32)
            d = lax.bitwise_and(_lsr(u, shift), _vec(_NUM_BINS - 1))
            occ, last = plsc.scan_count(d)
            lasti = lax.convert_element_type(last, jnp.int32)
            meta[pl.ds((j * _NSEG + s) * _L, _L)] = lax.bitwise_or(
                d,
                lax.bitwise_or(
                    lax.shift_left(occ, _vec(_RADIX_BITS)),
                    lax.shift_left(lasti, _vec(_RADIX_BITS + 5)),
                ),
            )

        def cbody(g, _):
          work = []
          for r in range(rounds_per_iter):
            j = g * rounds_per_iter + r
            for s in range(_NSEG):
              c = s * seg_chunks + wbase + j
              p = meta[pl.ds((j * _NSEG + s) * _L, _L)]
              if src is None:
                v_idx = iota + c * _L
              else:
                v_idx = src[pl.ds(c * _L, _L)]
              d = lax.bitwise_and(p, _vec(_NUM_BINS - 1))
              occ = lax.bitwise_and(_lsr(p, _RADIX_BITS), _vec(31))
              last = lax.ne(_lsr(p, _RADIX_BITS + 5), _vec(0))
              work.append((s, d, occ, last, v_idx))
          for s, d, occ, last, v_idx in work:
            base = plsc.load_gather(hsegs[s], [d])
            pos = base + occ
            plsc.store_scatter(hsegs[s], [d], pos, mask=last)
            plsc.store_scatter(dst, [pos], v_idx)
          return 0

        lax.fori_loop(0, win_rounds // rounds_per_iter, cbody, 0)

    def row_body(r, _):
      row = wid * rows_per_worker + r
      pltpu.sync_copy(x_hbm.at[row], keyf)
      zero_hists()
      transform_and_count0()
      prefix_hists()
      permute(None, ping, _SHIFTS[0])
      zero_hists()
      count_sweep(ping, _SHIFTS[1])
      prefix_hists()
      permute(ping, pong, _SHIFTS[1])
      zero_hists()
      count_sweep(pong, _SHIFTS[2])
      prefix_hists()
      permute(pong, ping, _SHIFTS[2])
      pltpu.sync_copy(ping, out_hbm.at[row])
      return 0

    lax.fori_loop(0, rows_per_worker, row_body, 0)

  run = pl.kernel(
      body,
      out_type=jax.ShapeDtypeStruct((rows, n), jnp.int32),
      mesh=mesh,
      compiler_params=pltpu.CompilerParams(needs_layout_passes=False),
      scratch_types=[
          pltpu.VMEM((n,), jnp.float32),
          pltpu.VMEM((n,), jnp.int32),
          pltpu.VMEM((n,), jnp.int32),
          pltpu.VMEM((512 * _L,), jnp.int32),
          pltpu.VMEM((_NUM_BINS,), jnp.int32),
          pltpu.VMEM((_NUM_BINS,), jnp.int32),
          pltpu.VMEM((_NUM_BINS,), jnp.int32),
          pltpu.VMEM((_NUM_BINS,), jnp.int32),
      ],
  )
  return run(x)


def kernel(x):
  return _argsort_rows(x)


# speedup vs baseline: 1.1351x; 1.1351x over previous
import functools

import jax
import jax.numpy as jnp
from jax import lax
from jax.experimental import pallas as pl
from jax.experimental.pallas import tpu as pltpu
from jax.experimental.pallas import tpu_sc as plsc

_NUM_CORES = 2
_NUM_SUBCORES = 16
_NUM_WORKERS = _NUM_CORES * _NUM_SUBCORES
_L = 16

_RADIX_BITS = 11
_NUM_BINS = 1 << _RADIX_BITS
_SHIFTS = (0, _RADIX_BITS, 2 * _RADIX_BITS)


def _vec(val):
  return lax.full((_L,), val, jnp.int32)


def _lsr(x, k):
  if k == 0:
    return x
  return lax.shift_right_logical(x, _vec(k))


def _to_sortable_bits(f):
  b = plsc.bitcast(f, jnp.int32)
  sgn = lax.shift_right_arithmetic(b, _vec(31))
  flip = lax.bitwise_or(sgn, _vec(-(2**31)))
  return lax.bitwise_xor(b, flip)


@functools.partial(jax.jit, static_argnames=())
def _argsort_rows(x):
  rows, n = x.shape
  assert rows % _NUM_WORKERS == 0 and n % _L == 0
  rows_per_worker = rows // _NUM_WORKERS
  num_chunks = n // _L
  hist_chunks = _NUM_BINS // _L

  mesh = plsc.VectorSubcoreMesh(
      core_axis_name="c", subcore_axis_name="s")

  def body(x_hbm, out_hbm, keyf, ping, pong, meta, hist0, hist1, hist2):
    cid = lax.axis_index("c")
    sid = lax.axis_index("s")
    wid = sid * _NUM_CORES + cid
    hists = (hist0, hist1, hist2)

    def zero_hists():
      @plsc.parallel_loop(0, hist_chunks, unroll=4)
      def _(j):
        sl = pl.ds(j * _L, _L)
        hist0[sl] = _vec(0)
        hist1[sl] = _vec(0)
        hist2[sl] = _vec(0)

    def transform_and_count():
      @plsc.parallel_loop(0, num_chunks, unroll=4)
      def _(j):
        sl = pl.ds(j * _L, _L)
        u = _to_sortable_bits(keyf[sl])
        keyf[sl] = plsc.bitcast(u, jnp.float32)
        for p in range(3):
          d = lax.bitwise_and(_lsr(u, _SHIFTS[p]), _vec(_NUM_BINS - 1))
          occ, last = plsc.scan_count(d)
          plsc.addupdate_scatter(hists[p], [d], occ, mask=last)

    def prefix_hist(h):
      def pbody(j, carry):
        v = h[pl.ds(j * _L, _L)]
        csum = plsc.cumsum(v)
        h[pl.ds(j * _L, _L)] = csum - v + carry
        return carry + jnp.sum(v)
      lax.fori_loop(0, hist_chunks, pbody, jnp.int32(-1))

    def meta_chunk(src, shift, j, c):
      sl = pl.ds(c * _L, _L)
      if src is None:
        u = plsc.bitcast(keyf[sl], jnp.int32)
      else:
        u = plsc.bitcast(plsc.load_gather(keyf, [src[sl]]), jnp.int32)
      d = lax.bitwise_and(_lsr(u, shift), _vec(_NUM_BINS - 1))
      occ, last = plsc.scan_count(d)
      lasti = lax.convert_element_type(last, jnp.int32)
      meta[pl.ds(j * _L, _L)] = lax.bitwise_or(
          d,
          lax.bitwise_or(
              lax.shift_left(occ, _vec(_RADIX_BITS)),
              lax.shift_left(lasti, _vec(_RADIX_BITS + 5)),
          ),
      )

    def permute(src, dst, shift, hist_cur):
      iota = lax.iota(jnp.int32, _L)
      half_chunks = num_chunks // 2
      group = 16

      for half in range(2):
        base_chunk = half * half_chunks

        @plsc.parallel_loop(0, half_chunks, unroll=8)
        def _(j):
          meta_chunk(src, shift, j, j + base_chunk)

        def cbody(g, _):
          regs = []
          for k in range(group):
            j = g * group + k
            c = j + base_chunk
            p = meta[pl.ds(j * _L, _L)]
            if src is None:
              v_idx = iota + c * _L
            else:
              v_idx = src[pl.ds(c * _L, _L)]
            d = lax.bitwise_and(p, _vec(_NUM_BINS - 1))
            occ = lax.bitwise_and(_lsr(p, _RADIX_BITS), _vec(31))
            last = lax.ne(_lsr(p, _RADIX_BITS + 5), _vec(0))
            regs.append((d, occ, last, v_idx))
          for d, occ, last, v_idx in regs:
            base = plsc.load_gather(hist_cur, [d])
            pos = base + occ
            plsc.store_scatter(hist_cur, [d], pos, mask=last)
            plsc.store_scatter(dst, [pos], v_idx)
          return 0

        lax.fori_loop(0, half_chunks // group, cbody, 0)

    def row_body(r, _):
      row = wid * rows_per_worker + r
      pltpu.sync_copy(x_hbm.at[row], keyf)
      zero_hists()
      transform_and_count()
      prefix_hist(hist0)
      permute(None, ping, _SHIFTS[0], hist0)
      prefix_hist(hist1)
      permute(ping, pong, _SHIFTS[1], hist1)
      prefix_hist(hist2)
      permute(pong, ping, _SHIFTS[2], hist2)
      pltpu.sync_copy(ping, out_hbm.at[row])
      return 0

    lax.fori_loop(0, rows_per_worker, row_body, 0)

  run = pl.kernel(
      body,
      out_type=jax.ShapeDtypeStruct((rows, n), jnp.int32),
      mesh=mesh,
      compiler_params=pltpu.CompilerParams(needs_layout_passes=False),
      scratch_types=[
          pltpu.VMEM((n,), jnp.float32),
          pltpu.VMEM((n,), jnp.int32),
          pltpu.VMEM((n,), jnp.int32),
          pltpu.VMEM((n // 2,), jnp.int32),
          pltpu.VMEM((_NUM_BINS,), jnp.int32),
          pltpu.VMEM((_NUM_BINS,), jnp.int32),
          pltpu.VMEM((_NUM_BINS,), jnp.int32),
      ],
  )
  return run(x)


def kernel(x):
  return _argsort_rows(x)
